# baseline (device time: 68830 ns/iter reference)
import jax
import jax.numpy as jnp
from jax import lax
from jax.experimental import pallas as pl
from jax.experimental.pallas import tpu as pltpu

N_RING = 8
M = 1024
BLK = M // N_RING


def _ring_coords(q):
    q = jnp.mod(q, N_RING)
    yq = jnp.where(q < 4, 0, 1).astype(jnp.int32)
    zq = jnp.where(q < 4, q, 7 - q).astype(jnp.int32)
    return yq, zq


def kernel(dy, W):
    m, f = dy.shape
    d = W.shape[0]

    x = lax.axis_index("x")
    y = lax.axis_index("y")
    z = lax.axis_index("z")
    p = jnp.where(y == 0, z, 7 - z).astype(jnp.int32)

    dy_blk = lax.dynamic_slice_in_dim(dy, p * BLK, BLK, axis=0)
    partial = jnp.einsum(
        "mf,df->md", dy_blk, W, preferred_element_type=jnp.float32
    )

    def body(partial_ref, out_ref, xbuf, x_sems, send_sems, recv_sems):
        my_x = lax.axis_index("x")
        my_y = lax.axis_index("y")
        my_z = lax.axis_index("z")
        my_p = jnp.where(my_y == 0, my_z, 7 - my_z).astype(jnp.int32)

        right_y, right_z = _ring_coords(my_p + 1)
        left_y, left_z = _ring_coords(my_p - 1)

        barrier_sem = pltpu.get_barrier_semaphore()
        for ty, tz in ((right_y, right_z), (left_y, left_z)):
            pl.semaphore_signal(
                barrier_sem, inc=1,
                device_id=(my_x, ty, tz),
                device_id_type=pl.DeviceIdType.MESH,
            )
        pl.semaphore_signal(
            barrier_sem, inc=1,
            device_id=(1 - my_x, my_y, my_z),
            device_id_type=pl.DeviceIdType.MESH,
        )
        pl.semaphore_wait(barrier_sem, 3)

        xr = pltpu.make_async_remote_copy(
            src_ref=partial_ref,
            dst_ref=xbuf,
            send_sem=x_sems.at[0],
            recv_sem=x_sems.at[1],
            device_id=(1 - my_x, my_y, my_z),
            device_id_type=pl.DeviceIdType.MESH,
        )
        xr.start()
        xr.wait()

        out_ref[pl.ds(my_p * BLK, BLK), :] = partial_ref[...] + xbuf[...]

        for h in range(N_RING - 1):
            o_send = jnp.mod(my_p - h, N_RING)
            o_recv = jnp.mod(my_p - h - 1, N_RING)
            send = pltpu.make_async_remote_copy(
                src_ref=out_ref.at[pl.ds(o_send * BLK, BLK), :],
                dst_ref=out_ref.at[pl.ds(o_send * BLK, BLK), :],
                send_sem=send_sems.at[h],
                recv_sem=recv_sems.at[h],
                device_id=(my_x, right_y, right_z),
                device_id_type=pl.DeviceIdType.MESH,
            )
            send.start()
            recv = pltpu.make_async_remote_copy(
                src_ref=out_ref.at[pl.ds(o_recv * BLK, BLK), :],
                dst_ref=out_ref.at[pl.ds(o_recv * BLK, BLK), :],
                send_sem=send_sems.at[h],
                recv_sem=recv_sems.at[h],
                device_id=(my_x, left_y, left_z),
                device_id_type=pl.DeviceIdType.MESH,
            )
            recv.wait_recv()
            send.wait_send()

    return pl.pallas_call(
        body,
        out_shape=jax.ShapeDtypeStruct((m, d), jnp.float32),
        in_specs=[pl.BlockSpec(memory_space=pltpu.VMEM)],
        out_specs=pl.BlockSpec(memory_space=pltpu.VMEM),
        scratch_shapes=[
            pltpu.VMEM((BLK, d), jnp.float32),
            pltpu.SemaphoreType.DMA((2,)),
            pltpu.SemaphoreType.DMA((N_RING - 1,)),
            pltpu.SemaphoreType.DMA((N_RING - 1,)),
        ],
        compiler_params=pltpu.CompilerParams(collective_id=0),
    )(partial)


# device time: 47531 ns/iter; 1.4481x vs baseline; 1.4481x over previous
import jax
import jax.numpy as jnp
from jax import lax
from jax.experimental import pallas as pl
from jax.experimental.pallas import tpu as pltpu

N_RING = 8
M = 1024
BLK = M // N_RING


def _ring_coords(q):
    q = jnp.mod(q, N_RING)
    yq = jnp.where(q < 4, 0, 1).astype(jnp.int32)
    zq = jnp.where(q < 4, q, 7 - q).astype(jnp.int32)
    return yq, zq


def kernel(dy, W):
    m, f = dy.shape
    d = W.shape[0]

    x = lax.axis_index("x")
    y = lax.axis_index("y")
    z = lax.axis_index("z")
    p = jnp.where(y == 0, z, 7 - z).astype(jnp.int32)

    dy_blk = lax.dynamic_slice_in_dim(dy, p * BLK, BLK, axis=0)
    partial = jnp.einsum(
        "mf,df->md", dy_blk, W, preferred_element_type=jnp.float32
    )

    def body(partial_ref, out_ref, xbuf, x_sems, sendR, recvR, sendL, recvL):
        my_x = lax.axis_index("x")
        my_y = lax.axis_index("y")
        my_z = lax.axis_index("z")
        my_p = jnp.where(my_y == 0, my_z, 7 - my_z).astype(jnp.int32)

        right_y, right_z = _ring_coords(my_p + 1)
        left_y, left_z = _ring_coords(my_p - 1)

        barrier_sem = pltpu.get_barrier_semaphore()
        for ty, tz in ((right_y, right_z), (left_y, left_z)):
            pl.semaphore_signal(
                barrier_sem, inc=1,
                device_id=(my_x, ty, tz),
                device_id_type=pl.DeviceIdType.MESH,
            )
        pl.semaphore_signal(
            barrier_sem, inc=1,
            device_id=(1 - my_x, my_y, my_z),
            device_id_type=pl.DeviceIdType.MESH,
        )
        pl.semaphore_wait(barrier_sem, 3)

        xr = pltpu.make_async_remote_copy(
            src_ref=partial_ref,
            dst_ref=xbuf,
            send_sem=x_sems.at[0],
            recv_sem=x_sems.at[1],
            device_id=(1 - my_x, my_y, my_z),
            device_id_type=pl.DeviceIdType.MESH,
        )
        xr.start()
        xr.wait()

        out_ref[pl.ds(my_p * BLK, BLK), :] = partial_ref[...] + xbuf[...]

        N_R, N_L = 4, 3

        def mk(o_blk, sem_s, sem_r, hop, to_y, to_z):
            return pltpu.make_async_remote_copy(
                src_ref=out_ref.at[pl.ds(o_blk * BLK, BLK), :],
                dst_ref=out_ref.at[pl.ds(o_blk * BLK, BLK), :],
                send_sem=sem_s.at[hop],
                recv_sem=sem_r.at[hop],
                device_id=(my_x, to_y, to_z),
                device_id_type=pl.DeviceIdType.MESH,
            )

        def send_r(hop):
            o = jnp.mod(my_p - hop, N_RING)
            rd = mk(o, sendR, recvR, hop, right_y, right_z)
            rd.start()
            return rd

        def recv_r(hop):
            o = jnp.mod(my_p - hop - 1, N_RING)
            mk(o, sendR, recvR, hop, left_y, left_z).wait_recv()

        def send_l(hop):
            o = jnp.mod(my_p + hop, N_RING)
            rd = mk(o, sendL, recvL, hop, left_y, left_z)
            rd.start()
            return rd

        def recv_l(hop):
            o = jnp.mod(my_p + hop + 1, N_RING)
            mk(o, sendL, recvL, hop, right_y, right_z).wait_recv()

        sends = [send_r(0), send_l(0)]
        for h in range(max(N_R, N_L) - 1):
            if h < N_R - 1:
                recv_r(h)
                sends.append(send_r(h + 1))
            if h < N_L - 1:
                recv_l(h)
                sends.append(send_l(h + 1))
        recv_l(N_L - 1)
        recv_r(N_R - 1)
        for rd in sends:
            rd.wait_send()

    return pl.pallas_call(
        body,
        out_shape=jax.ShapeDtypeStruct((m, d), jnp.float32),
        in_specs=[pl.BlockSpec(memory_space=pltpu.VMEM)],
        out_specs=pl.BlockSpec(memory_space=pltpu.VMEM),
        scratch_shapes=[
            pltpu.VMEM((BLK, d), jnp.float32),
            pltpu.SemaphoreType.DMA((2,)),
            pltpu.SemaphoreType.DMA((4,)),
            pltpu.SemaphoreType.DMA((4,)),
            pltpu.SemaphoreType.DMA((3,)),
            pltpu.SemaphoreType.DMA((3,)),
        ],
        compiler_params=pltpu.CompilerParams(collective_id=0),
    )(partial)


# device time: 40528 ns/iter; 1.6983x vs baseline; 1.1728x over previous
import jax
import jax.numpy as jnp
from jax import lax
from jax.experimental import pallas as pl
from jax.experimental.pallas import tpu as pltpu

N_RING = 8
M = 1024
D = 1024
BLK = M // N_RING
HALF = D // 2
N_R, N_L = 4, 3


def _ring_coords(q):
    q = jnp.mod(q, N_RING)
    yq = jnp.where(q < 4, 0, 1).astype(jnp.int32)
    zq = jnp.where(q < 4, q, 7 - q).astype(jnp.int32)
    return yq, zq


def kernel(dy, W):
    m, f = dy.shape
    d = W.shape[0]

    y = lax.axis_index("y")
    z = lax.axis_index("z")
    p = jnp.where(y == 0, z, 7 - z).astype(jnp.int32)

    dy_blk = lax.dynamic_slice_in_dim(dy, p * BLK, BLK, axis=0)
    partial = jnp.einsum(
        "mf,df->md", dy_blk, W, preferred_element_type=jnp.float32
    )

    def body(partial_ref, out_ref, xbuf,
             rs_sems, sendR, recvR, sendL, recvL, xsend, xrecv):
        my_x = lax.axis_index("x")
        my_y = lax.axis_index("y")
        my_z = lax.axis_index("z")
        my_p = jnp.where(my_y == 0, my_z, 7 - my_z).astype(jnp.int32)

        right_y, right_z = _ring_coords(my_p + 1)
        left_y, left_z = _ring_coords(my_p - 1)
        my_col = my_x * HALF
        other_col = (1 - my_x) * HALF

        barrier_sem = pltpu.get_barrier_semaphore()
        for ty, tz in ((right_y, right_z), (left_y, left_z)):
            pl.semaphore_signal(
                barrier_sem, inc=1,
                device_id=(my_x, ty, tz),
                device_id_type=pl.DeviceIdType.MESH,
            )
        pl.semaphore_signal(
            barrier_sem, inc=1,
            device_id=(1 - my_x, my_y, my_z),
            device_id_type=pl.DeviceIdType.MESH,
        )
        pl.semaphore_wait(barrier_sem, 3)

        rs = pltpu.make_async_remote_copy(
            src_ref=partial_ref.at[:, pl.ds(other_col, HALF)],
            dst_ref=xbuf,
            send_sem=rs_sems.at[0],
            recv_sem=rs_sems.at[1],
            device_id=(1 - my_x, my_y, my_z),
            device_id_type=pl.DeviceIdType.MESH,
        )
        rs.start()
        rs.wait()

        out_ref[pl.ds(my_p * BLK, BLK), pl.ds(my_col, HALF)] = (
            partial_ref[:, pl.ds(my_col, HALF)] + xbuf[...]
        )

        def xfwd(o_blk, idx):
            rd = pltpu.make_async_remote_copy(
                src_ref=out_ref.at[pl.ds(o_blk * BLK, BLK),
                                   pl.ds(my_col, HALF)],
                dst_ref=out_ref.at[pl.ds(o_blk * BLK, BLK),
                                   pl.ds(my_col, HALF)],
                send_sem=xsend.at[idx],
                recv_sem=xrecv.at[idx],
                device_id=(1 - my_x, my_y, my_z),
                device_id_type=pl.DeviceIdType.MESH,
            )
            rd.start()
            return rd

        def mk(o_blk, sem_s, sem_r, hop, to_y, to_z):
            return pltpu.make_async_remote_copy(
                src_ref=out_ref.at[pl.ds(o_blk * BLK, BLK),
                                   pl.ds(my_col, HALF)],
                dst_ref=out_ref.at[pl.ds(o_blk * BLK, BLK),
                                   pl.ds(my_col, HALF)],
                send_sem=sem_s.at[hop],
                recv_sem=sem_r.at[hop],
                device_id=(my_x, to_y, to_z),
                device_id_type=pl.DeviceIdType.MESH,
            )

        def send_r(hop):
            o = jnp.mod(my_p - hop, N_RING)
            rd = mk(o, sendR, recvR, hop, right_y, right_z)
            rd.start()
            return rd

        def recv_r(hop):
            o = jnp.mod(my_p - hop - 1, N_RING)
            mk(o, sendR, recvR, hop, left_y, left_z).wait_recv()
            return o

        def send_l(hop):
            o = jnp.mod(my_p + hop, N_RING)
            rd = mk(o, sendL, recvL, hop, left_y, left_z)
            rd.start()
            return rd

        def recv_l(hop):
            o = jnp.mod(my_p + hop + 1, N_RING)
            mk(o, sendL, recvL, hop, right_y, right_z).wait_recv()
            return o

        sends = [xfwd(my_p, 0), send_r(0), send_l(0)]
        for h in range(max(N_R, N_L)):
            if h < N_R:
                o = recv_r(h)
                if h < N_R - 1:
                    sends.append(send_r(h + 1))
                sends.append(xfwd(o, 1 + h))
            if h < N_L:
                o = recv_l(h)
                if h < N_L - 1:
                    sends.append(send_l(h + 1))
                sends.append(xfwd(o, 1 + N_R + h))

        for idx in range(1 + N_R + N_L):
            rd = pltpu.make_async_remote_copy(
                src_ref=out_ref.at[pl.ds(0, BLK), pl.ds(my_col, HALF)],
                dst_ref=out_ref.at[pl.ds(0, BLK), pl.ds(my_col, HALF)],
                send_sem=xsend.at[idx],
                recv_sem=xrecv.at[idx],
                device_id=(1 - my_x, my_y, my_z),
                device_id_type=pl.DeviceIdType.MESH,
            )
            rd.wait_recv()
        for rd in sends:
            rd.wait_send()

    return pl.pallas_call(
        body,
        out_shape=jax.ShapeDtypeStruct((m, d), jnp.float32),
        in_specs=[pl.BlockSpec(memory_space=pltpu.VMEM)],
        out_specs=pl.BlockSpec(memory_space=pltpu.VMEM),
        scratch_shapes=[
            pltpu.VMEM((BLK, HALF), jnp.float32),
            pltpu.SemaphoreType.DMA((2,)),
            pltpu.SemaphoreType.DMA((N_R,)),
            pltpu.SemaphoreType.DMA((N_R,)),
            pltpu.SemaphoreType.DMA((N_L,)),
            pltpu.SemaphoreType.DMA((N_L,)),
            pltpu.SemaphoreType.DMA((8,)),
            pltpu.SemaphoreType.DMA((8,)),
        ],
        compiler_params=pltpu.CompilerParams(collective_id=0),
    )(partial)
